# baseline (device time: 368454 ns/iter reference)
import jax
import jax.numpy as jnp
from jax import lax
from jax.experimental import pallas as pl
from jax.experimental.pallas import tpu as pltpu

M = 32768
N = 1024
QROWS = M // 4
C = 16
CH = QROWS // C
FD = 6
FX = 5
FZ = C - FD - FX


def kernel(x):
    def body(
        x_ref, out_ref, xs_ref, p1_ref, red_ref, s1_ref, s2_ref,
        xsd_sem, xqd_sem, p1s, p1r, gxs, gxr, gzs, gzr,
        gyds, gydr, fxs, fxr, fzs, fzr, loc_sem,
    ):
        xi = lax.axis_index("x")
        yi = lax.axis_index("y")
        zi = lax.axis_index("z")
        q = 2 * ((yi + zi) % 2) + (xi + zi) % 2
        qx = q ^ 1
        qy = q ^ 2
        qz = q ^ 3
        ynbr = (xi, 1 - yi, zi)
        xnbr = (1 - xi, yi, zi)
        znbr = (xi, yi, 1 - zi)

        bsem = pltpu.get_barrier_semaphore()
        for nbr in (ynbr, xnbr, znbr):
            pl.semaphore_signal(
                bsem, inc=1, device_id=nbr, device_id_type=pl.DeviceIdType.MESH
            )
        pl.semaphore_wait(bsem, 3)

        xsd = [
            pltpu.make_async_copy(
                x_ref.at[pl.ds(qy * QROWS + c * CH, CH)],
                s1_ref.at[c % 2],
                xsd_sem.at[c],
            )
            for c in range(C)
        ]
        xqd = [
            pltpu.make_async_copy(
                x_ref.at[pl.ds(q * QROWS + c * CH, CH)],
                s2_ref.at[c % 2],
                xqd_sem.at[c],
            )
            for c in range(C)
        ]
        p1 = [
            pltpu.make_async_remote_copy(
                src_ref=xs_ref.at[pl.ds(c * CH, CH)],
                dst_ref=p1_ref.at[pl.ds(c * CH, CH)],
                send_sem=p1s.at[c],
                recv_sem=p1r.at[c],
                device_id=ynbr,
                device_id_type=pl.DeviceIdType.MESH,
            )
            for c in range(C)
        ]

        xsd[0].start()
        xsd[1].start()
        xqd[0].start()
        xqd[1].start()

        gx = []
        gz = []
        gyd = []

        def do_add(c):
            xqd[c].wait()
            p1[c].wait()
            sl = pl.ds(c * CH, CH)
            red_ref[sl] = s2_ref[c % 2].astype(jnp.bfloat16) + p1_ref[sl]
            if c + 2 < C:
                xqd[c + 2].start()
            osl = pl.ds(q * QROWS + c * CH, CH)
            gxc = pltpu.make_async_remote_copy(
                src_ref=red_ref.at[sl],
                dst_ref=out_ref.at[osl],
                send_sem=gxs.at[c],
                recv_sem=gxr.at[c],
                device_id=xnbr,
                device_id_type=pl.DeviceIdType.MESH,
            )
            gxc.start()
            gx.append(gxc)
            gzc = pltpu.make_async_remote_copy(
                src_ref=red_ref.at[sl],
                dst_ref=out_ref.at[osl],
                send_sem=gzs.at[c],
                recv_sem=gzr.at[c],
                device_id=znbr,
                device_id_type=pl.DeviceIdType.MESH,
            )
            gzc.start()
            gz.append(gzc)
            if c < FD:
                gydc = pltpu.make_async_remote_copy(
                    src_ref=red_ref.at[sl],
                    dst_ref=out_ref.at[osl],
                    send_sem=gyds.at[c],
                    recv_sem=gydr.at[c],
                    device_id=ynbr,
                    device_id_type=pl.DeviceIdType.MESH,
                )
                gydc.start()
                gyd.append(gydc)

        K = 2
        for c in range(C):
            xsd[c].wait()
            xs_ref[pl.ds(c * CH, CH)] = s1_ref[c % 2].astype(jnp.bfloat16)
            if c + 2 < C:
                xsd[c + 2].start()
            p1[c].start()
            if c >= K:
                do_add(c - K)
        for c in range(C - K, C):
            do_add(c)

        loc = pltpu.make_async_copy(
            red_ref, out_ref.at[pl.ds(q * QROWS, QROWS)], loc_sem
        )
        loc.start()

        fx = []
        for k in range(FX):
            c = FD + k
            gz[c].wait()
            osl = pl.ds(qz * QROWS + c * CH, CH)
            fc = pltpu.make_async_remote_copy(
                src_ref=out_ref.at[osl],
                dst_ref=out_ref.at[osl],
                send_sem=fxs.at[k],
                recv_sem=fxr.at[k],
                device_id=xnbr,
                device_id_type=pl.DeviceIdType.MESH,
            )
            fc.start()
            fx.append(fc)

        fz = []
        for j in range(FZ):
            c = FD + FX + j
            gx[c].wait()
            osl = pl.ds(qx * QROWS + c * CH, CH)
            fc = pltpu.make_async_remote_copy(
                src_ref=out_ref.at[osl],
                dst_ref=out_ref.at[osl],
                send_sem=fzs.at[j],
                recv_sem=fzr.at[j],
                device_id=znbr,
                device_id_type=pl.DeviceIdType.MESH,
            )
            fc.start()
            fz.append(fc)

        for c in range(C):
            if c < FD + FX:
                gx[c].wait()
            if not (FD <= c < FD + FX):
                gz[c].wait()
            if c < FD:
                gyd[c].wait()
        for k in range(FX):
            fx[k].wait()
        for j in range(FZ):
            fz[j].wait()
        loc.wait()

    return pl.pallas_call(
        body,
        out_shape=jax.ShapeDtypeStruct((M, N), jnp.bfloat16),
        in_specs=[pl.BlockSpec(memory_space=pltpu.MemorySpace.HBM)],
        out_specs=pl.BlockSpec(memory_space=pltpu.MemorySpace.HBM),
        scratch_shapes=[
            pltpu.VMEM((QROWS, N), jnp.bfloat16),
            pltpu.VMEM((QROWS, N), jnp.bfloat16),
            pltpu.VMEM((QROWS, N), jnp.bfloat16),
            pltpu.VMEM((2, CH, N), jnp.float32),
            pltpu.VMEM((2, CH, N), jnp.float32),
            pltpu.SemaphoreType.DMA((C,)),
            pltpu.SemaphoreType.DMA((C,)),
            pltpu.SemaphoreType.DMA((C,)),
            pltpu.SemaphoreType.DMA((C,)),
            pltpu.SemaphoreType.DMA((C,)),
            pltpu.SemaphoreType.DMA((C,)),
            pltpu.SemaphoreType.DMA((C,)),
            pltpu.SemaphoreType.DMA((C,)),
            pltpu.SemaphoreType.DMA((FD,)),
            pltpu.SemaphoreType.DMA((FD,)),
            pltpu.SemaphoreType.DMA((FX,)),
            pltpu.SemaphoreType.DMA((FX,)),
            pltpu.SemaphoreType.DMA((FZ,)),
            pltpu.SemaphoreType.DMA((FZ,)),
            pltpu.SemaphoreType.DMA,
        ],
        compiler_params=pltpu.CompilerParams(
            collective_id=0, vmem_limit_bytes=64 * 1024 * 1024
        ),
    )(x)


# device time: 367603 ns/iter; 1.0023x vs baseline; 1.0023x over previous
import jax
import jax.numpy as jnp
from jax import lax
from jax.experimental import pallas as pl
from jax.experimental.pallas import tpu as pltpu

M = 32768
N = 1024
QROWS = M // 4
C = 16
CH = QROWS // C
FD = 6
FX = 5
FZ = C - FD - FX


def kernel(x):
    def body(
        x_ref, out_ref, xs_ref, p1_ref, red_ref, s1_ref, s2_ref,
        xsd_sem, xqd_sem, p1s, p1r, gxs, gxr, gzs, gzr,
        gyds, gydr, fxs, fxr, fzs, fzr, loc_sem,
    ):
        xi = lax.axis_index("x")
        yi = lax.axis_index("y")
        zi = lax.axis_index("z")
        q = 2 * ((yi + zi) % 2) + (xi + zi) % 2
        qx = q ^ 1
        qy = q ^ 2
        qz = q ^ 3
        ynbr = (xi, 1 - yi, zi)
        xnbr = (1 - xi, yi, zi)
        znbr = (xi, yi, 1 - zi)

        bsem = pltpu.get_barrier_semaphore()
        for nbr in (ynbr, xnbr, znbr):
            pl.semaphore_signal(
                bsem, inc=1, device_id=nbr, device_id_type=pl.DeviceIdType.MESH
            )
        pl.semaphore_wait(bsem, 3)

        xsd = [
            pltpu.make_async_copy(
                x_ref.at[pl.ds(qy * QROWS + c * CH, CH)],
                s1_ref.at[c % 2],
                xsd_sem.at[c],
            )
            for c in range(C)
        ]
        xqd = [
            pltpu.make_async_copy(
                x_ref.at[pl.ds(q * QROWS + c * CH, CH)],
                s2_ref.at[c % 2],
                xqd_sem.at[c],
            )
            for c in range(C)
        ]
        p1 = [
            pltpu.make_async_remote_copy(
                src_ref=xs_ref.at[pl.ds(c * CH, CH)],
                dst_ref=p1_ref.at[pl.ds(c * CH, CH)],
                send_sem=p1s.at[c],
                recv_sem=p1r.at[c],
                device_id=ynbr,
                device_id_type=pl.DeviceIdType.MESH,
            )
            for c in range(C)
        ]

        xsd[0].start()
        xsd[1].start()
        xqd[0].start()
        xqd[1].start()

        gx = []
        gz = []
        gyd = []

        def do_add(c):
            xqd[c].wait()
            p1[c].wait()
            sl = pl.ds(c * CH, CH)
            red_ref[sl] = s2_ref[c % 2].astype(jnp.bfloat16) + p1_ref[sl]
            if c + 2 < C:
                xqd[c + 2].start()
            osl = pl.ds(q * QROWS + c * CH, CH)
            gxc = pltpu.make_async_remote_copy(
                src_ref=red_ref.at[sl],
                dst_ref=out_ref.at[osl],
                send_sem=gxs.at[c],
                recv_sem=gxr.at[c],
                device_id=xnbr,
                device_id_type=pl.DeviceIdType.MESH,
            )
            gxc.start()
            gx.append(gxc)
            gzc = pltpu.make_async_remote_copy(
                src_ref=red_ref.at[sl],
                dst_ref=out_ref.at[osl],
                send_sem=gzs.at[c],
                recv_sem=gzr.at[c],
                device_id=znbr,
                device_id_type=pl.DeviceIdType.MESH,
            )
            gzc.start()
            gz.append(gzc)
            if c < FD:
                gydc = pltpu.make_async_remote_copy(
                    src_ref=red_ref.at[sl],
                    dst_ref=out_ref.at[osl],
                    send_sem=gyds.at[c],
                    recv_sem=gydr.at[c],
                    device_id=ynbr,
                    device_id_type=pl.DeviceIdType.MESH,
                )
                gydc.start()
                gyd.append(gydc)

        K = 4
        for c in range(C):
            xsd[c].wait()
            xs_ref[pl.ds(c * CH, CH)] = s1_ref[c % 2].astype(jnp.bfloat16)
            if c + 2 < C:
                xsd[c + 2].start()
            p1[c].start()
            if c >= K:
                do_add(c - K)
        for c in range(C - K, C):
            do_add(c)

        loc = pltpu.make_async_copy(
            red_ref, out_ref.at[pl.ds(q * QROWS, QROWS)], loc_sem
        )
        loc.start()

        fx = []
        for k in range(FX):
            c = FD + k
            gz[c].wait()
            osl = pl.ds(qz * QROWS + c * CH, CH)
            fc = pltpu.make_async_remote_copy(
                src_ref=out_ref.at[osl],
                dst_ref=out_ref.at[osl],
                send_sem=fxs.at[k],
                recv_sem=fxr.at[k],
                device_id=xnbr,
                device_id_type=pl.DeviceIdType.MESH,
            )
            fc.start()
            fx.append(fc)

        fz = []
        for j in range(FZ):
            c = FD + FX + j
            gx[c].wait()
            osl = pl.ds(qx * QROWS + c * CH, CH)
            fc = pltpu.make_async_remote_copy(
                src_ref=out_ref.at[osl],
                dst_ref=out_ref.at[osl],
                send_sem=fzs.at[j],
                recv_sem=fzr.at[j],
                device_id=znbr,
                device_id_type=pl.DeviceIdType.MESH,
            )
            fc.start()
            fz.append(fc)

        for c in range(C):
            if c < FD + FX:
                gx[c].wait()
            if not (FD <= c < FD + FX):
                gz[c].wait()
            if c < FD:
                gyd[c].wait()
        for k in range(FX):
            fx[k].wait()
        for j in range(FZ):
            fz[j].wait()
        loc.wait()

    return pl.pallas_call(
        body,
        out_shape=jax.ShapeDtypeStruct((M, N), jnp.bfloat16),
        in_specs=[pl.BlockSpec(memory_space=pltpu.MemorySpace.HBM)],
        out_specs=pl.BlockSpec(memory_space=pltpu.MemorySpace.HBM),
        scratch_shapes=[
            pltpu.VMEM((QROWS, N), jnp.bfloat16),
            pltpu.VMEM((QROWS, N), jnp.bfloat16),
            pltpu.VMEM((QROWS, N), jnp.bfloat16),
            pltpu.VMEM((2, CH, N), jnp.float32),
            pltpu.VMEM((2, CH, N), jnp.float32),
            pltpu.SemaphoreType.DMA((C,)),
            pltpu.SemaphoreType.DMA((C,)),
            pltpu.SemaphoreType.DMA((C,)),
            pltpu.SemaphoreType.DMA((C,)),
            pltpu.SemaphoreType.DMA((C,)),
            pltpu.SemaphoreType.DMA((C,)),
            pltpu.SemaphoreType.DMA((C,)),
            pltpu.SemaphoreType.DMA((C,)),
            pltpu.SemaphoreType.DMA((FD,)),
            pltpu.SemaphoreType.DMA((FD,)),
            pltpu.SemaphoreType.DMA((FX,)),
            pltpu.SemaphoreType.DMA((FX,)),
            pltpu.SemaphoreType.DMA((FZ,)),
            pltpu.SemaphoreType.DMA((FZ,)),
            pltpu.SemaphoreType.DMA,
        ],
        compiler_params=pltpu.CompilerParams(
            collective_id=0, vmem_limit_bytes=64 * 1024 * 1024
        ),
    )(x)


# device time: 300148 ns/iter; 1.2276x vs baseline; 1.2247x over previous
import jax
import jax.numpy as jnp
from jax import lax
from jax.experimental import pallas as pl
from jax.experimental.pallas import tpu as pltpu

M = 32768
N = 1024
QROWS = M // 4
C = 16
CH = QROWS // C
FD = 6
FX = 5
FZ = C - FD - FX


def kernel(x):
    def body(
        x_ref, out_ref, xs_ref, p1_ref, red_ref, s1_ref, s2_ref,
        xsd_sem, xqd_sem, p1s, p1r, gxs, gxr, gzs, gzr,
        gyds, gydr, fxs, fxr, fzs, fzr, loc_sem,
    ):
        xi = lax.axis_index("x")
        yi = lax.axis_index("y")
        zi = lax.axis_index("z")
        q = 2 * ((yi + zi) % 2) + (xi + zi) % 2
        qx = q ^ 1
        qy = q ^ 2
        qz = q ^ 3
        ynbr = (xi, 1 - yi, zi)
        xnbr = (1 - xi, yi, zi)
        znbr = (xi, yi, 1 - zi)

        bsem = pltpu.get_barrier_semaphore()
        for nbr in (ynbr, xnbr, znbr):
            pl.semaphore_signal(
                bsem, inc=1, device_id=nbr, device_id_type=pl.DeviceIdType.MESH
            )
        pl.semaphore_wait(bsem, 3)

        xsd = [
            pltpu.make_async_copy(
                x_ref.at[pl.ds(qy * QROWS + c * CH, CH)],
                s1_ref.at[c % 2],
                xsd_sem.at[c],
            )
            for c in range(C)
        ]
        xqd = [
            pltpu.make_async_copy(
                x_ref.at[pl.ds(q * QROWS + c * CH, CH)],
                s2_ref.at[c % 2],
                xqd_sem.at[c],
            )
            for c in range(C)
        ]
        p1 = [
            pltpu.make_async_remote_copy(
                src_ref=xs_ref.at[pl.ds(c * CH, CH)],
                dst_ref=p1_ref.at[pl.ds(c * CH, CH)],
                send_sem=p1s.at[c],
                recv_sem=p1r.at[c],
                device_id=ynbr,
                device_id_type=pl.DeviceIdType.MESH,
            )
            for c in range(C)
        ]

        xsd[0].start()
        xsd[1].start()
        xqd[0].start()
        xqd[1].start()

        gx = []
        gz = []
        gyd = []

        def do_add(c):
            xqd[c].wait()
            p1[c].wait()
            sl = pl.ds(c * CH, CH)
            red_ref[sl] = s2_ref[c % 2].astype(jnp.bfloat16) + p1_ref[sl]
            if c + 2 < C:
                xqd[c + 2].start()
            osl = pl.ds(q * QROWS + c * CH, CH)
            gxc = pltpu.make_async_remote_copy(
                src_ref=red_ref.at[sl],
                dst_ref=out_ref.at[osl],
                send_sem=gxs.at[c],
                recv_sem=gxr.at[c],
                device_id=xnbr,
                device_id_type=pl.DeviceIdType.MESH,
            )
            gxc.start()
            gx.append(gxc)
            gzc = pltpu.make_async_remote_copy(
                src_ref=red_ref.at[sl],
                dst_ref=out_ref.at[osl],
                send_sem=gzs.at[c],
                recv_sem=gzr.at[c],
                device_id=znbr,
                device_id_type=pl.DeviceIdType.MESH,
            )
            gzc.start()
            gz.append(gzc)

        K = 4
        for c in range(C):
            xsd[c].wait()
            xs_ref[pl.ds(c * CH, CH)] = s1_ref[c % 2].astype(jnp.bfloat16)
            if c + 2 < C:
                xsd[c + 2].start()
            p1[c].start()
            if c >= K:
                do_add(c - K)

        for c in range(FD):
            sl = pl.ds(c * CH, CH)
            osl = pl.ds(q * QROWS + c * CH, CH)
            gydc = pltpu.make_async_remote_copy(
                src_ref=red_ref.at[sl],
                dst_ref=out_ref.at[osl],
                send_sem=gyds.at[c],
                recv_sem=gydr.at[c],
                device_id=ynbr,
                device_id_type=pl.DeviceIdType.MESH,
            )
            gydc.start()
            gyd.append(gydc)

        for c in range(C - K, C):
            do_add(c)

        loc = pltpu.make_async_copy(
            red_ref, out_ref.at[pl.ds(q * QROWS, QROWS)], loc_sem
        )
        loc.start()

        fx = []
        for k in range(FX):
            c = FD + k
            gz[c].wait()
            osl = pl.ds(qz * QROWS + c * CH, CH)
            fc = pltpu.make_async_remote_copy(
                src_ref=out_ref.at[osl],
                dst_ref=out_ref.at[osl],
                send_sem=fxs.at[k],
                recv_sem=fxr.at[k],
                device_id=xnbr,
                device_id_type=pl.DeviceIdType.MESH,
            )
            fc.start()
            fx.append(fc)

        fz = []
        for j in range(FZ):
            c = FD + FX + j
            gx[c].wait()
            osl = pl.ds(qx * QROWS + c * CH, CH)
            fc = pltpu.make_async_remote_copy(
                src_ref=out_ref.at[osl],
                dst_ref=out_ref.at[osl],
                send_sem=fzs.at[j],
                recv_sem=fzr.at[j],
                device_id=znbr,
                device_id_type=pl.DeviceIdType.MESH,
            )
            fc.start()
            fz.append(fc)

        for c in range(C):
            if c < FD + FX:
                gx[c].wait()
            if not (FD <= c < FD + FX):
                gz[c].wait()
            if c < FD:
                gyd[c].wait()
        for k in range(FX):
            fx[k].wait()
        for j in range(FZ):
            fz[j].wait()
        loc.wait()

    return pl.pallas_call(
        body,
        out_shape=jax.ShapeDtypeStruct((M, N), jnp.bfloat16),
        in_specs=[pl.BlockSpec(memory_space=pltpu.MemorySpace.HBM)],
        out_specs=pl.BlockSpec(memory_space=pltpu.MemorySpace.HBM),
        scratch_shapes=[
            pltpu.VMEM((QROWS, N), jnp.bfloat16),
            pltpu.VMEM((QROWS, N), jnp.bfloat16),
            pltpu.VMEM((QROWS, N), jnp.bfloat16),
            pltpu.VMEM((2, CH, N), jnp.float32),
            pltpu.VMEM((2, CH, N), jnp.float32),
            pltpu.SemaphoreType.DMA((C,)),
            pltpu.SemaphoreType.DMA((C,)),
            pltpu.SemaphoreType.DMA((C,)),
            pltpu.SemaphoreType.DMA((C,)),
            pltpu.SemaphoreType.DMA((C,)),
            pltpu.SemaphoreType.DMA((C,)),
            pltpu.SemaphoreType.DMA((C,)),
            pltpu.SemaphoreType.DMA((C,)),
            pltpu.SemaphoreType.DMA((FD,)),
            pltpu.SemaphoreType.DMA((FD,)),
            pltpu.SemaphoreType.DMA((FX,)),
            pltpu.SemaphoreType.DMA((FX,)),
            pltpu.SemaphoreType.DMA((FZ,)),
            pltpu.SemaphoreType.DMA((FZ,)),
            pltpu.SemaphoreType.DMA,
        ],
        compiler_params=pltpu.CompilerParams(
            collective_id=0, vmem_limit_bytes=64 * 1024 * 1024
        ),
    )(x)


# device time: 299829 ns/iter; 1.2289x vs baseline; 1.0011x over previous
import jax
import jax.numpy as jnp
from jax import lax
from jax.experimental import pallas as pl
from jax.experimental.pallas import tpu as pltpu

M = 32768
N = 1024
QROWS = M // 4
C = 16
CH = QROWS // C
FD = 6
FX = 5
FZ = C - FD - FX


def kernel(x):
    def body(
        x_ref, out_ref, xs_ref, p1_ref, red_ref, s1_ref, s2_ref,
        xsd_sem, xqd_sem, p1s, p1r, gxs, gxr, gzs, gzr,
        gyds, gydr, fxs, fxr, fzs, fzr, loc_sem,
    ):
        xi = lax.axis_index("x")
        yi = lax.axis_index("y")
        zi = lax.axis_index("z")
        q = 2 * ((yi + zi) % 2) + (xi + zi) % 2
        qx = q ^ 1
        qy = q ^ 2
        qz = q ^ 3
        ynbr = (xi, 1 - yi, zi)
        xnbr = (1 - xi, yi, zi)
        znbr = (xi, yi, 1 - zi)

        xsd = [
            pltpu.make_async_copy(
                x_ref.at[pl.ds(qy * QROWS + c * CH, CH)],
                s1_ref.at[c % 2],
                xsd_sem.at[c],
            )
            for c in range(C)
        ]
        xqd = [
            pltpu.make_async_copy(
                x_ref.at[pl.ds(q * QROWS + c * CH, CH)],
                s2_ref.at[c % 2],
                xqd_sem.at[c],
            )
            for c in range(C)
        ]
        p1 = [
            pltpu.make_async_remote_copy(
                src_ref=xs_ref.at[pl.ds(c * CH, CH)],
                dst_ref=p1_ref.at[pl.ds(c * CH, CH)],
                send_sem=p1s.at[c],
                recv_sem=p1r.at[c],
                device_id=ynbr,
                device_id_type=pl.DeviceIdType.MESH,
            )
            for c in range(C)
        ]

        xsd[0].start()
        xsd[1].start()
        xqd[0].start()
        xqd[1].start()

        bsem = pltpu.get_barrier_semaphore()
        for nbr in (ynbr, xnbr, znbr):
            pl.semaphore_signal(
                bsem, inc=1, device_id=nbr, device_id_type=pl.DeviceIdType.MESH
            )
        pl.semaphore_wait(bsem, 3)

        gx = []
        gz = []
        gyd = []

        def do_add(c):
            xqd[c].wait()
            p1[c].wait()
            sl = pl.ds(c * CH, CH)
            red_ref[sl] = s2_ref[c % 2].astype(jnp.bfloat16) + p1_ref[sl]
            if c + 2 < C:
                xqd[c + 2].start()
            osl = pl.ds(q * QROWS + c * CH, CH)
            gxc = pltpu.make_async_remote_copy(
                src_ref=red_ref.at[sl],
                dst_ref=out_ref.at[osl],
                send_sem=gxs.at[c],
                recv_sem=gxr.at[c],
                device_id=xnbr,
                device_id_type=pl.DeviceIdType.MESH,
            )
            gxc.start()
            gx.append(gxc)
            gzc = pltpu.make_async_remote_copy(
                src_ref=red_ref.at[sl],
                dst_ref=out_ref.at[osl],
                send_sem=gzs.at[c],
                recv_sem=gzr.at[c],
                device_id=znbr,
                device_id_type=pl.DeviceIdType.MESH,
            )
            gzc.start()
            gz.append(gzc)

        K = 4
        for c in range(C):
            xsd[c].wait()
            xs_ref[pl.ds(c * CH, CH)] = s1_ref[c % 2].astype(jnp.bfloat16)
            if c + 2 < C:
                xsd[c + 2].start()
            p1[c].start()
            if c >= K:
                do_add(c - K)

        for c in range(FD):
            sl = pl.ds(c * CH, CH)
            osl = pl.ds(q * QROWS + c * CH, CH)
            gydc = pltpu.make_async_remote_copy(
                src_ref=red_ref.at[sl],
                dst_ref=out_ref.at[osl],
                send_sem=gyds.at[c],
                recv_sem=gydr.at[c],
                device_id=ynbr,
                device_id_type=pl.DeviceIdType.MESH,
            )
            gydc.start()
            gyd.append(gydc)

        for c in range(C - K, C):
            do_add(c)

        loc = pltpu.make_async_copy(
            red_ref, out_ref.at[pl.ds(q * QROWS, QROWS)], loc_sem
        )
        loc.start()

        fx = []
        for k in range(FX):
            c = FD + k
            gz[c].wait()
            osl = pl.ds(qz * QROWS + c * CH, CH)
            fc = pltpu.make_async_remote_copy(
                src_ref=out_ref.at[osl],
                dst_ref=out_ref.at[osl],
                send_sem=fxs.at[k],
                recv_sem=fxr.at[k],
                device_id=xnbr,
                device_id_type=pl.DeviceIdType.MESH,
            )
            fc.start()
            fx.append(fc)

        fz = []
        for j in range(FZ):
            c = FD + FX + j
            gx[c].wait()
            osl = pl.ds(qx * QROWS + c * CH, CH)
            fc = pltpu.make_async_remote_copy(
                src_ref=out_ref.at[osl],
                dst_ref=out_ref.at[osl],
                send_sem=fzs.at[j],
                recv_sem=fzr.at[j],
                device_id=znbr,
                device_id_type=pl.DeviceIdType.MESH,
            )
            fc.start()
            fz.append(fc)

        for c in range(C):
            if c < FD + FX:
                gx[c].wait()
            if not (FD <= c < FD + FX):
                gz[c].wait()
            if c < FD:
                gyd[c].wait()
        for k in range(FX):
            fx[k].wait()
        for j in range(FZ):
            fz[j].wait()
        loc.wait()

    return pl.pallas_call(
        body,
        out_shape=jax.ShapeDtypeStruct((M, N), jnp.bfloat16),
        in_specs=[pl.BlockSpec(memory_space=pltpu.MemorySpace.HBM)],
        out_specs=pl.BlockSpec(memory_space=pltpu.MemorySpace.HBM),
        scratch_shapes=[
            pltpu.VMEM((QROWS, N), jnp.bfloat16),
            pltpu.VMEM((QROWS, N), jnp.bfloat16),
            pltpu.VMEM((QROWS, N), jnp.bfloat16),
            pltpu.VMEM((2, CH, N), jnp.float32),
            pltpu.VMEM((2, CH, N), jnp.float32),
            pltpu.SemaphoreType.DMA((C,)),
            pltpu.SemaphoreType.DMA((C,)),
            pltpu.SemaphoreType.DMA((C,)),
            pltpu.SemaphoreType.DMA((C,)),
            pltpu.SemaphoreType.DMA((C,)),
            pltpu.SemaphoreType.DMA((C,)),
            pltpu.SemaphoreType.DMA((C,)),
            pltpu.SemaphoreType.DMA((C,)),
            pltpu.SemaphoreType.DMA((FD,)),
            pltpu.SemaphoreType.DMA((FD,)),
            pltpu.SemaphoreType.DMA((FX,)),
            pltpu.SemaphoreType.DMA((FX,)),
            pltpu.SemaphoreType.DMA((FZ,)),
            pltpu.SemaphoreType.DMA((FZ,)),
            pltpu.SemaphoreType.DMA,
        ],
        compiler_params=pltpu.CompilerParams(
            collective_id=0, vmem_limit_bytes=64 * 1024 * 1024
        ),
    )(x)
